# Initial kernel scaffold; baseline (speedup 1.0000x reference)
import jax
import jax.numpy as jnp
from jax.experimental import pallas as pl


def kernel(x, edge_index, W_gcn, b_gcn, W1, b1, W2, b2, W3, b3):
    raise NotImplementedError("write your pallas kernel here")


# speedup vs baseline: 27.8274x; 27.8274x over previous
import functools

import jax
import jax.numpy as jnp
from jax import lax
from jax.experimental import pallas as pl
from jax.experimental.pallas import tpu as pltpu, tpu_sc as plsc

N = 10000
E = 320000
D = 128
H = 64
C = 4

NC = 2
NS = 16
NW = NC * NS
NP = 10112
RPS = NP // NS

CH = 128
NCH = 79
EPW = NCH * CH
EP = NW * EPW
DW = 16

_mesh = plsc.VectorSubcoreMesh(core_axis_name="c", subcore_axis_name="s")
_sc_params = pltpu.CompilerParams(use_tc_tiling_on_sc=False)


@functools.partial(
    pl.kernel,
    out_type=jax.ShapeDtypeStruct((NC, NP, DW), jnp.float32),
    mesh=_mesh,
    compiler_params=_sc_params,
    scratch_types=[
        pltpu.VMEM((NCH, CH), jnp.int32),
        pltpu.VMEM((CH, DW), jnp.float32),
        pltpu.VMEM_SHARED((NP, DW), jnp.float32),
    ],
)
def _deg_kernel(dst_hbm, ones_hbm, zero_hbm, out_hbm, dst_v, ones_v, deg_sh):
  c = lax.axis_index("c")
  s = lax.axis_index("s")
  wid = s * NC + c
  row0 = pl.multiple_of(s * RPS, 8)

  pltpu.sync_copy(zero_hbm.at[pl.ds(row0, RPS)], deg_sh.at[pl.ds(row0, RPS)])
  pltpu.sync_copy(dst_hbm.at[wid], dst_v)
  pltpu.sync_copy(ones_hbm, ones_v)
  plsc.subcore_barrier()

  @pl.loop(0, NCH)
  def _(j):
    pltpu.sync_copy(ones_v, deg_sh.at[dst_v.at[j]], add=True)

  plsc.subcore_barrier()
  pltpu.sync_copy(deg_sh.at[pl.ds(row0, RPS)],
                  out_hbm.at[c, pl.ds(row0, RPS)])


@functools.partial(
    pl.kernel,
    out_type=jax.ShapeDtypeStruct((NC, NP, H), jnp.float32),
    mesh=_mesh,
    compiler_params=_sc_params,
    scratch_types=[
        pltpu.VMEM((NCH, CH), jnp.int32),
        pltpu.VMEM((NCH, CH), jnp.int32),
        pltpu.VMEM((CH, H), jnp.float32),
        pltpu.VMEM_SHARED((NP, H), jnp.float32),
        pltpu.SemaphoreType.DMA,
    ],
)
def _msg_kernel(y_hbm, src_hbm, dst_hbm, zero_hbm, out_hbm,
                src_v, dst_v, rows_v, z_sh, sem):
  c = lax.axis_index("c")
  s = lax.axis_index("s")
  wid = s * NC + c
  row0 = pl.multiple_of(s * RPS, 8)

  pltpu.sync_copy(zero_hbm.at[pl.ds(row0, RPS)], z_sh.at[pl.ds(row0, RPS)])
  pltpu.sync_copy(src_hbm.at[wid], src_v)
  pltpu.sync_copy(dst_hbm.at[wid], dst_v)
  plsc.subcore_barrier()

  @pl.loop(0, NCH)
  def _(j):
    pltpu.async_copy(y_hbm.at[src_v.at[j]], rows_v, sem).wait()
    pltpu.sync_copy(rows_v, z_sh.at[dst_v.at[j]], add=True)

  plsc.subcore_barrier()
  pltpu.sync_copy(z_sh.at[pl.ds(row0, RPS)],
                  out_hbm.at[c, pl.ds(row0, RPS)])


def _mm_body(x_ref, w_ref, deg_ref, y_ref):
  deg = deg_ref[0, :, 0] + deg_ref[1, :, 0] + 1.0
  dinv = lax.rsqrt(deg)
  xw = jnp.dot(x_ref[...], w_ref[...], preferred_element_type=jnp.float32)
  y_ref[...] = xw * dinv[:, None]


def _mm_call(x_p, w, deg_parts):
  return pl.pallas_call(
      _mm_body,
      out_shape=jax.ShapeDtypeStruct((NP, H), jnp.float32),
  )(x_p, w, deg_parts)


def _head_body(z_ref, y_ref, deg_ref, bg_ref, w1_ref, b1_ref, w2_ref, b2_ref,
               w3_ref, b3_ref, o_ref):
  deg = deg_ref[0, :, 0] + deg_ref[1, :, 0] + 1.0
  dinv = lax.rsqrt(deg)
  z = z_ref[0] + z_ref[1] + y_ref[...]
  h = jax.nn.relu(z * dinv[:, None] + bg_ref[...])
  h = jax.nn.relu(
      jnp.dot(h, w1_ref[...], preferred_element_type=jnp.float32) + b1_ref[...])
  h = jax.nn.relu(
      jnp.dot(h, w2_ref[...], preferred_element_type=jnp.float32) + b2_ref[...])
  h = jnp.dot(h, w3_ref[...], preferred_element_type=jnp.float32) + b3_ref[...]
  m = jnp.max(h, axis=1, keepdims=True)
  lse = jnp.log(jnp.sum(jnp.exp(h - m), axis=1, keepdims=True))
  o_ref[...] = h - m - lse


def _head_call(z_parts, y, deg_parts, bg, w1, b1, w2, b2, w3, b3):
  return pl.pallas_call(
      _head_body,
      out_shape=jax.ShapeDtypeStruct((NP, C), jnp.float32),
  )(z_parts, y, deg_parts, bg, w1, b1, w2, b2, w3, b3)


def kernel(x, edge_index, W_gcn, b_gcn, W1, b1, W2, b2, W3, b3):
  src = edge_index[0]
  dst = edge_index[1]

  pad = jnp.full((EP - E,), N, dtype=jnp.int32)
  src_p = jnp.concatenate([src, pad]).reshape(NW, NCH, CH)
  dst_p = jnp.concatenate([dst, pad]).reshape(NW, NCH, CH)
  x_p = jnp.pad(x, ((0, NP - N), (0, 0)))
  zeros = jnp.zeros((NP, H), jnp.float32)
  ones_rows = jnp.ones((CH, DW), jnp.float32)
  zeros_rows = jnp.zeros((NP, DW), jnp.float32)

  deg_parts = _deg_kernel(dst_p, ones_rows, zeros_rows)
  y = _mm_call(x_p, W_gcn, deg_parts)
  z_parts = _msg_kernel(y, src_p, dst_p, zeros)
  out = _head_call(z_parts, y, deg_parts,
                   b_gcn.reshape(1, H), W1, b1.reshape(1, 32),
                   W2, b2.reshape(1, 16), W3, b3.reshape(1, C))
  return out[:N]
